# baseline (speedup 1.0000x reference)
import functools

import jax
import jax.numpy as jnp
from jax.experimental import pallas as pl
from jax.experimental.pallas import tpu as pltpu

_BLOCK_R = 8


def _kl_block_kernel(k_ref, e_ref, out_ref):
    e = e_ref[...]
    k = k_ref[...]
    m_e = jnp.max(e, axis=1, keepdims=True)
    ee = jnp.exp(e - m_e)
    z_e = jnp.sum(ee, axis=1, keepdims=True)
    s = jnp.sum(ee * (e - k), axis=1, keepdims=True)
    m_k = jnp.max(k, axis=1, keepdims=True)
    z_k = jnp.sum(jnp.exp(k - m_k), axis=1, keepdims=True)
    row = s / z_e - m_e - jnp.log(z_e) + m_k + jnp.log(z_k)
    part = jnp.sum(row).reshape(1, 1)

    @pl.when(pl.program_id(0) == 0)
    def _():
        out_ref[...] = jnp.zeros_like(out_ref)

    out_ref[...] += part


@functools.partial(jax.jit, static_argnames=())
def kernel(korean_rep, english_rep):
    n_rows, vocab = korean_rep.shape
    grid = (n_rows // _BLOCK_R,)
    out = pl.pallas_call(
        _kl_block_kernel,
        grid=grid,
        in_specs=[
            pl.BlockSpec((_BLOCK_R, vocab), lambda i: (i, 0)),
            pl.BlockSpec((_BLOCK_R, vocab), lambda i: (i, 0)),
        ],
        out_specs=pl.BlockSpec((1, 1), lambda i: (0, 0)),
        out_shape=jax.ShapeDtypeStruct((1, 1), jnp.float32),
        compiler_params=pltpu.CompilerParams(
            dimension_semantics=("arbitrary",),
        ),
    )(korean_rep, english_rep)
    return out[0, 0] / n_rows


# speedup vs baseline: 1.1197x; 1.1197x over previous
import functools

import jax
import jax.numpy as jnp
from jax.experimental import pallas as pl
from jax.experimental.pallas import tpu as pltpu

_BLOCK_R = 16


def _kl_block_kernel(k_ref, e_ref, out_ref):
    e = e_ref[...]
    k = k_ref[...]
    m_e = jnp.max(e, axis=1, keepdims=True)
    ee = jnp.exp(e - m_e)
    z_e = jnp.sum(ee, axis=1, keepdims=True)
    s = jnp.sum(ee * (e - k), axis=1, keepdims=True)
    m_k = jnp.max(k, axis=1, keepdims=True)
    z_k = jnp.sum(jnp.exp(k - m_k), axis=1, keepdims=True)
    row = s / z_e - m_e - jnp.log(z_e) + m_k + jnp.log(z_k)
    part = jnp.sum(row).reshape(1, 1)

    @pl.when(pl.program_id(0) == 0)
    def _():
        out_ref[...] = jnp.zeros_like(out_ref)

    out_ref[...] += part


@functools.partial(jax.jit, static_argnames=())
def kernel(korean_rep, english_rep):
    n_rows, vocab = korean_rep.shape
    grid = (n_rows // _BLOCK_R,)
    out = pl.pallas_call(
        _kl_block_kernel,
        grid=grid,
        in_specs=[
            pl.BlockSpec((_BLOCK_R, vocab), lambda i: (i, 0)),
            pl.BlockSpec((_BLOCK_R, vocab), lambda i: (i, 0)),
        ],
        out_specs=pl.BlockSpec((1, 1), lambda i: (0, 0)),
        out_shape=jax.ShapeDtypeStruct((1, 1), jnp.float32),
        compiler_params=pltpu.CompilerParams(
            dimension_semantics=("arbitrary",),
        ),
    )(korean_rep, english_rep)
    return out[0, 0] / n_rows
